# baseline (speedup 1.0000x reference)
import functools

import jax
import jax.numpy as jnp
from jax import lax
from jax.experimental import pallas as pl
from jax.experimental.pallas import tpu as pltpu
from jax.experimental.pallas import tpu_sc as plsc

_N = 10000
_E = 320000
_ED = 16
_HID = 64
_H = 2
_NCLS = 30
_G = 16
_HD = _H * _HID

_K = 64
_NCHUNK = _E // _K
_NTILE = 16
_ITERS = -(-_NCHUNK // _NTILE)
_NB = 4
_DACC = 96
_NPAD = 10112
_ROWS_PER_TILE = _NPAD // _NTILE

_EXP_SCATTER = True
_EXP_COMPUTE = True

_SC_PARAMS = pltpu.CompilerParams(
    use_tc_tiling_on_sc=False, needs_layout_passes=False)
_MESH = plsc.VectorSubcoreMesh(core_axis_name="c", subcore_axis_name="s")


def _proj_body(x_ref, wq, bq, wk, bk, wv, bv, we, ws, bs,
               kv_ref, qq_ref, skip_ref):
    x = x_ref[...]
    q = jnp.dot(x, wq[...].T, preferred_element_type=jnp.float32) + bq[0]
    kv_ref[0, :, :_HID] = (jnp.dot(x, wk[...].T,
                                   preferred_element_type=jnp.float32) + bk[0])
    kv_ref[0, :, _HID:] = (jnp.dot(x, wv[...].T,
                                   preferred_element_type=jnp.float32) + bv[0])
    qq_ref[0, :, :_HID] = q
    qq_ref[0, :, _HID:] = jnp.dot(q, we[...],
                                  preferred_element_type=jnp.float32)
    skip_ref[0] = (jnp.dot(x, ws[...].T,
                           preferred_element_type=jnp.float32) + bs[0])


def _proj_call(x, wq, bq, wk, bk, wv, bv, we, ws, bs):
    blk = 2000
    d_in = x.shape[1]
    wspec = pl.BlockSpec((_HID, d_in), lambda i, h: (h, 0))
    bspec = pl.BlockSpec((1, 1, _HID), lambda i, h: (h, 0, 0))
    return pl.pallas_call(
        _proj_body,
        grid=(_N // blk, _H),
        in_specs=[pl.BlockSpec((blk, d_in), lambda i, h: (i, 0)),
                  wspec, bspec, wspec, bspec, wspec, bspec,
                  pl.BlockSpec((_HID, _ED), lambda i, h: (h, 0)),
                  wspec, bspec],
        out_specs=[pl.BlockSpec((1, blk, _HD), lambda i, h: (h, i, 0)),
                   pl.BlockSpec((1, blk, _HID + _ED), lambda i, h: (h, i, 0)),
                   pl.BlockSpec((1, blk, _HID), lambda i, h: (h, i, 0))],
        out_shape=[jax.ShapeDtypeStruct((_H, _N, _HD), jnp.float32),
                   jax.ShapeDtypeStruct((_H, _N, _HID + _ED), jnp.float32),
                   jax.ShapeDtypeStruct((_H, _N, _HID), jnp.float32)],
    )(x, wq, bq, wk, bk, wv, bv, we, ws, bs)


def _edge_sc_body(src_hbm, dst_hbm, ea_hbm, kv_hbm, qq_hbm,
                  acc_hbm,
                  src_v, dst_v, kv_v, qq_v, ea_v, orow_v, dsc_v, acc_sh,
                  gsem, isem, ssem):
    cid = lax.axis_index("c")
    sid = lax.axis_index("s")
    base = cid * _N

    zero16 = jnp.zeros((16,), jnp.float32)
    lane = jnp.arange(16, dtype=jnp.int32)

    def _zrow(r, carry):
        for par in range(2):
            for col in range(_DACC // 16):
                orow_v[par, r, pl.ds(col * 16, 16)] = zero16
        return carry
    lax.fori_loop(0, _K, _zrow, 0)
    tile0 = sid * _ROWS_PER_TILE
    for i in range(9):
        pltpu.sync_copy(orow_v.at[0], acc_sh.at[pl.ds(tile0 + i * _K, _K)])
    pltpu.sync_copy(orow_v.at[0, pl.ds(0, 56)],
                    acc_sh.at[pl.ds(tile0 + 9 * _K, 56)])
    plsc.subcore_barrier()

    def _adjust_src(b):
        for t in range(_K // 16):
            src_v[b, pl.ds(t * 16, 16)] = src_v[b, pl.ds(t * 16, 16)] + base

    def _issue(b, c):
        off = c * _K
        pltpu.async_copy(kv_hbm.at[src_v.at[b]], kv_v.at[b], gsem)
        pltpu.async_copy(qq_hbm.at[dst_v.at[b]], qq_v.at[b], gsem)
        pltpu.async_copy(ea_hbm.at[pl.ds(off, _K)], ea_v.at[b], gsem)

    def _wait_gathers(b):
        pltpu.make_async_copy(kv_hbm.at[src_v.at[b]], kv_v.at[b], gsem).wait()
        pltpu.make_async_copy(qq_hbm.at[dst_v.at[b]], qq_v.at[b], gsem).wait()
        pltpu.make_async_copy(ea_hbm.at[pl.ds(0, _K)], ea_v.at[b], gsem).wait()

    def _issue_idx(b, c):
        off = c * _K
        pltpu.async_copy(src_hbm.at[pl.ds(off, _K)], src_v.at[b], isem)
        pltpu.async_copy(dst_hbm.at[pl.ds(off, _K)], dst_v.at[b], isem)

    def _wait_idx(b):
        pltpu.make_async_copy(src_hbm.at[pl.ds(0, _K)], src_v.at[b], isem).wait()
        pltpu.make_async_copy(dst_hbm.at[pl.ds(0, _K)], dst_v.at[b], isem).wait()

    def _wait_scatter(p):
        pltpu.make_async_copy(orow_v.at[p], acc_sh.at[dsc_v.at[p]], ssem).wait()

    def _copy_dst(b, p):
        for t in range(_K // 16):
            dsc_v[p, pl.ds(t * 16, 16)] = dst_v[b, pl.ds(t * 16, 16)]

    def _compute(b, p):
        bsp = jnp.full((16,), b, jnp.int32)
        psp = jnp.full((16,), p, jnp.int32)
        cvec = lambda d: jnp.broadcast_to(d, (16,)).astype(jnp.int32)

        def _group(g, carry):
            rows = lane + g * 16

            def _dot4(d4, accs):
                out = []
                for u in range(4):
                    d = d4 * 4 + u
                    c = cvec(d)
                    out.append(accs[u] +
                               plsc.load_gather(qq_v, [bsp, rows, c]) *
                               plsc.load_gather(kv_v, [bsp, rows, c]))
                return tuple(out)

            accs = lax.fori_loop(0, _HID // 4, _dot4, (zero16,) * 4)

            def _dote4(d4, accs):
                out = []
                for u in range(4):
                    d = d4 * 4 + u
                    out.append(accs[u] +
                               plsc.load_gather(qq_v, [bsp, rows, cvec(_HID + d)]) *
                               plsc.load_gather(ea_v, [bsp, rows, cvec(d)]))
                return tuple(out)

            accs = lax.fori_loop(0, _ED // 4, _dote4, accs)
            ex = jnp.exp(((accs[0] + accs[1]) + (accs[2] + accs[3])) * 0.125)

            def _scale4(d4, carry2):
                for u in range(4):
                    d = d4 * 4 + u
                    vcol = plsc.load_gather(kv_v, [bsp, rows, cvec(_HID + d)])
                    plsc.store_scatter(orow_v, [psp, rows, cvec(d)], ex * vcol)
                return carry2

            lax.fori_loop(0, _HID // 4, _scale4, 0)

            def _scalee4(d4, carry2):
                for u in range(4):
                    d = d4 * 4 + u
                    eacol = plsc.load_gather(ea_v, [bsp, rows, cvec(d)])
                    plsc.store_scatter(orow_v, [psp, rows, cvec(_HID + d)],
                                       ex * eacol)
                return carry2

            lax.fori_loop(0, _ED // 4, _scalee4, 0)
            plsc.store_scatter(orow_v, [psp, rows, cvec(80)], ex)
            return carry
        lax.fori_loop(0, _K // 16, _group, 0)

    for b in range(_NB - 1):
        cb = b * _NTILE + sid
        pltpu.sync_copy(src_hbm.at[pl.ds(cb * _K, _K)], src_v.at[b])
        pltpu.sync_copy(dst_hbm.at[pl.ds(cb * _K, _K)], dst_v.at[b])
        _adjust_src(b)
        _issue(b, cb)
    _issue_idx(_NB - 1, (_NB - 1) * _NTILE + sid)

    def _outer(k2, carry):
        for b in range(_NB):
            k = k2 * _NB + b
            c = k * _NTILE + sid
            bl = (b - 1) % _NB
            cl = c + (_NB - 1) * _NTILE
            p = b & 1

            @pl.when(cl < _NCHUNK)
            def _():
                _wait_idx(bl)
                _adjust_src(bl)
                _issue(bl, cl)

            @pl.when(c < _NCHUNK)
            def _():
                _wait_gathers(b)

                if _EXP_SCATTER:
                    @pl.when(k >= 2)
                    def _():
                        _wait_scatter(p)
                    _copy_dst(b, p)
                if _EXP_COMPUTE:
                    _compute(b, p)
                if _EXP_SCATTER:
                    pltpu.async_copy(orow_v.at[p], acc_sh.at[dsc_v.at[p]],
                                     ssem, add=True)

                @pl.when(c + _NB * _NTILE < _NCHUNK)
                def _():
                    _issue_idx(b, c + _NB * _NTILE)
        return carry

    lax.fori_loop(0, -(-_ITERS // _NB), _outer, 0)
    if _EXP_SCATTER:
        _wait_scatter(0)
        _wait_scatter(1)
    plsc.subcore_barrier()

    for i in range(9):
        rows = pl.ds(tile0 + i * _K, _K)
        pltpu.sync_copy(acc_sh.at[rows], acc_hbm.at[cid, rows])
    rows = pl.ds(tile0 + 9 * _K, 56)
    pltpu.sync_copy(acc_sh.at[rows], acc_hbm.at[cid, rows])


_edge_call = functools.partial(
    pl.kernel,
    mesh=_MESH,
    compiler_params=_SC_PARAMS,
    out_type=jax.ShapeDtypeStruct((2, _NPAD, _DACC), jnp.float32),
    scratch_types=[
        pltpu.VMEM((_NB, _K), jnp.int32),
        pltpu.VMEM((_NB, _K), jnp.int32),
        pltpu.VMEM((_NB, _K, _HD), jnp.float32),
        pltpu.VMEM((_NB, _K, _HID + _ED), jnp.float32),
        pltpu.VMEM((_NB, _K, _ED), jnp.float32),
        pltpu.VMEM((2, _K, _DACC), jnp.float32),
        pltpu.VMEM((2, _K), jnp.int32),
        pltpu.VMEM_SHARED((_NPAD, _DACC), jnp.float32),
        pltpu.SemaphoreType.DMA,
        pltpu.SemaphoreType.DMA,
        pltpu.SemaphoreType.DMA,
    ],
)(_edge_sc_body)


def _norm_h(acc, we_v, skip):
    out = []
    for h in range(_H):
        den = acc[h, :, 80:81] + 1e-16
        out.append((acc[h, :, :_HID] +
                    jnp.dot(acc[h, :, _HID:_HID + _ED],
                            we_v[h * _HID:(h + 1) * _HID, :].T,
                            preferred_element_type=jnp.float32)) / den
                   + skip[h])
    return jnp.maximum(jnp.concatenate(out, axis=1), 0.0)


def _post_body(acc_ref, skip_ref, we_ref, h_ref):
    h_ref[...] = _norm_h(acc_ref[...], we_ref[...], skip_ref[...])


def _post_call(acc, skip, we):
    blk = 2000
    return pl.pallas_call(
        _post_body,
        grid=(_N // blk,),
        in_specs=[pl.BlockSpec((2, blk, _DACC), lambda i: (0, i, 0)),
                  pl.BlockSpec((2, blk, _HID), lambda i: (0, i, 0)),
                  pl.BlockSpec(we.shape, lambda i: (0, 0))],
        out_specs=pl.BlockSpec((blk, _HD), lambda i: (i, 0)),
        out_shape=jax.ShapeDtypeStruct((_N, _HD), jnp.float32),
    )(acc, skip, we)


def _head_body(acc_ref, skip_ref, we_ref, batch_ref,
               wc1, bc1, wc2, bc2, out_ref):
    h = _norm_h(acc_ref[...], we_ref[...], skip_ref[...])

    b = batch_ref[...]
    gid = lax.broadcasted_iota(jnp.int32, (_G, _N), 0)
    m = jnp.where(b == gid, 1.0, 0.0)
    sums = jnp.dot(m, h, preferred_element_type=jnp.float32)
    cnt = jnp.sum(m, axis=1, keepdims=True)
    pooled = sums / jnp.maximum(cnt, 1.0)
    hid = jnp.maximum(jnp.dot(pooled, wc1[...].T,
                              preferred_element_type=jnp.float32) + bc1[...], 0.0)
    out_ref[...] = jnp.dot(hid, wc2[...].T,
                           preferred_element_type=jnp.float32) + bc2[...]


def _head_call(acc, skip, we, batch2d, wc1, bc1, wc2, bc2):
    return pl.pallas_call(
        _head_body,
        out_shape=jax.ShapeDtypeStruct((_G, _NCLS), jnp.float32),
    )(acc, skip, we, batch2d, wc1, bc1, wc2, bc2)


def _layer(h, src, dst, edge_attr, wq, bq, wk, bk, wv, bv, we, ws, bs):
    kv, qq, skip = _proj_call(h, wq, bq, wk, bk, wv, bv, we, ws, bs)
    kv2 = kv.reshape(_H * _N, _HD)
    qq2 = qq.reshape(_H * _N, _HID + _ED)
    acc = _edge_call(src, dst, edge_attr, kv2, qq2)
    return acc[:, :_N], skip


def kernel(x, edge_index, edge_attr, batch,
           Wq1, bq1, Wk1, bk1, Wv1, bv1, We1, Ws1, bs1,
           Wq2, bq2, Wk2, bk2, Wv2, bv2, We2, Ws2, bs2,
           Wc1, bc1, Wc2, bc2):
    src = edge_index[0].astype(jnp.int32)
    dst = edge_index[1].astype(jnp.int32)
    r2 = lambda b: b.reshape(1, -1)
    rb = lambda b: b.reshape(_H, 1, _HID)

    acc1, skip1 = _layer(x, src, dst, edge_attr,
                         Wq1, rb(bq1), Wk1, rb(bk1), Wv1, rb(bv1),
                         We1, Ws1, rb(bs1))
    h1 = _post_call(acc1, skip1, We1)
    acc2, skip2 = _layer(h1, src, dst, edge_attr,
                         Wq2, rb(bq2), Wk2, rb(bk2), Wv2, rb(bv2),
                         We2, Ws2, rb(bs2))
    return _head_call(acc2, skip2, We2,
                      batch.astype(jnp.int32).reshape(1, -1),
                      Wc1, r2(bc1), Wc2, r2(bc2))


# speedup vs baseline: 2.9136x; 2.9136x over previous
import functools

import jax
import jax.numpy as jnp
from jax import lax
from jax.experimental import pallas as pl
from jax.experimental.pallas import tpu as pltpu
from jax.experimental.pallas import tpu_sc as plsc

_N = 10000
_E = 320000
_ED = 16
_HID = 64
_H = 2
_NCLS = 30
_G = 16
_HD = _H * _HID

_K = 64
_NCHUNK = _E // _K
_NTILE = 16
_ITERS = -(-_NCHUNK // _NTILE)
_NB = 4
_DACC = 96
_ROWS_PER_TILE = 624

_EXP_SCATTER = True
_EXP_COMPUTE = True

_SC_PARAMS = pltpu.CompilerParams(
    use_tc_tiling_on_sc=False, needs_layout_passes=False)
_MESH = plsc.VectorSubcoreMesh(core_axis_name="c", subcore_axis_name="s")


def _proj_body(x_ref, wq, bq, wk, bk, wv, bv, we, ws, bs,
               kv_ref, qq_ref, skip_ref):
    x = x_ref[...]
    q = jnp.dot(x, wq[...].T, preferred_element_type=jnp.float32) + bq[0]
    kv_ref[0, :, :_HID] = (jnp.dot(x, wk[...].T,
                                   preferred_element_type=jnp.float32) + bk[0])
    kv_ref[0, :, _HID:] = (jnp.dot(x, wv[...].T,
                                   preferred_element_type=jnp.float32) + bv[0])
    qq_ref[0, :, :_HID] = q
    qq_ref[0, :, _HID:] = jnp.dot(q, we[...],
                                  preferred_element_type=jnp.float32)
    skip_ref[0] = (jnp.dot(x, ws[...].T,
                           preferred_element_type=jnp.float32) + bs[0])


def _proj_call(x, wq, bq, wk, bk, wv, bv, we, ws, bs):
    blk = 2000
    d_in = x.shape[1]
    wspec = pl.BlockSpec((_HID, d_in), lambda i, h: (h, 0))
    bspec = pl.BlockSpec((1, 1, _HID), lambda i, h: (h, 0, 0))
    return pl.pallas_call(
        _proj_body,
        grid=(_N // blk, _H),
        in_specs=[pl.BlockSpec((blk, d_in), lambda i, h: (i, 0)),
                  wspec, bspec, wspec, bspec, wspec, bspec,
                  pl.BlockSpec((_HID, _ED), lambda i, h: (h, 0)),
                  wspec, bspec],
        out_specs=[pl.BlockSpec((1, blk, _HD), lambda i, h: (h, i, 0)),
                   pl.BlockSpec((1, blk, _HID + _ED), lambda i, h: (h, i, 0)),
                   pl.BlockSpec((1, blk, _HID), lambda i, h: (h, i, 0))],
        out_shape=[jax.ShapeDtypeStruct((_H, _N, _HD), jnp.float32),
                   jax.ShapeDtypeStruct((_H, _N, _HID + _ED), jnp.float32),
                   jax.ShapeDtypeStruct((_H, _N, _HID), jnp.float32)],
    )(x, wq, bq, wk, bk, wv, bv, we, ws, bs)


def _edge_sc_body(src_hbm, dst_hbm, ea_hbm, kv_hbm, qq_hbm,
                  acc_hbm,
                  src_v, dst_v, kv_v, qq_v, ea_v, orow_v, dsc_v, stage_v,
                  acc_sh, gsem, isem, ssem):
    cid = lax.axis_index("c")
    sid = lax.axis_index("s")
    base = cid * _N

    zero16 = jnp.zeros((16,), jnp.float32)
    lane = jnp.arange(16, dtype=jnp.int32)

    def _zrow(r, carry):
        for par in range(2):
            for col in range(_DACC // 16):
                orow_v[par, r, pl.ds(col * 16, 16)] = zero16
        return carry
    lax.fori_loop(0, _K, _zrow, 0)
    tile0 = sid * _ROWS_PER_TILE
    for i in range(9):
        pltpu.sync_copy(orow_v.at[0], acc_sh.at[pl.ds(tile0 + i * _K, _K)])

    @pl.when(sid < 15)
    def _():
        pltpu.sync_copy(orow_v.at[0, pl.ds(0, 48)],
                        acc_sh.at[pl.ds(tile0 + 9 * _K, 48)])

    @pl.when(sid == 15)
    def _():
        pltpu.sync_copy(orow_v.at[0], acc_sh.at[pl.ds(tile0 + 9 * _K, _K)])
    plsc.subcore_barrier()

    def _adjust_src(b):
        for t in range(_K // 16):
            src_v[b, pl.ds(t * 16, 16)] = src_v[b, pl.ds(t * 16, 16)] + base

    def _issue(b, c):
        off = c * _K
        pltpu.async_copy(kv_hbm.at[src_v.at[b]], kv_v.at[b], gsem)
        pltpu.async_copy(qq_hbm.at[dst_v.at[b]], qq_v.at[b], gsem)
        pltpu.async_copy(ea_hbm.at[pl.ds(off, _K)], ea_v.at[b], gsem)

    def _wait_gathers(b):
        pltpu.make_async_copy(kv_hbm.at[src_v.at[b]], kv_v.at[b], gsem).wait()
        pltpu.make_async_copy(qq_hbm.at[dst_v.at[b]], qq_v.at[b], gsem).wait()
        pltpu.make_async_copy(ea_hbm.at[pl.ds(0, _K)], ea_v.at[b], gsem).wait()

    def _issue_idx(b, c):
        off = c * _K
        pltpu.async_copy(src_hbm.at[pl.ds(off, _K)], src_v.at[b], isem)
        pltpu.async_copy(dst_hbm.at[pl.ds(off, _K)], dst_v.at[b], isem)

    def _wait_idx(b):
        pltpu.make_async_copy(src_hbm.at[pl.ds(0, _K)], src_v.at[b], isem).wait()
        pltpu.make_async_copy(dst_hbm.at[pl.ds(0, _K)], dst_v.at[b], isem).wait()

    def _wait_scatter(p):
        pltpu.make_async_copy(orow_v.at[p], acc_sh.at[dsc_v.at[p]], ssem).wait()

    def _copy_dst(b, p):
        for t in range(_K // 16):
            dsc_v[p, pl.ds(t * 16, 16)] = dst_v[b, pl.ds(t * 16, 16)]

    def _compute(b, p):
        def _group(g, carry):
            jbase = g * 16
            for jj in range(16):
                j = jbase + jj
                s = qq_v[b, j, pl.ds(0, 16)] * kv_v[b, j, pl.ds(0, 16)]
                for t in range(1, 4):
                    s = s + (qq_v[b, j, pl.ds(t * 16, 16)] *
                             kv_v[b, j, pl.ds(t * 16, 16)])
                s = s + qq_v[b, j, pl.ds(64, 16)] * ea_v[b, j, :]
                stage_v[jj, pl.ds(0, 16)] = s

            accs = [None] * 4
            for c in range(16):
                col = plsc.load_gather(
                    stage_v, [lane, jnp.full((16,), c, jnp.int32)])
                accs[c % 4] = col if accs[c % 4] is None else accs[c % 4] + col
            ex = jnp.exp(((accs[0] + accs[1]) + (accs[2] + accs[3])) * 0.125)

            for jj in range(16):
                j = jbase + jj
                exsp = jnp.broadcast_to(ex[jj], (16,))
                for t in range(4):
                    orow_v[p, j, pl.ds(t * 16, 16)] = (
                        exsp * kv_v[b, j, pl.ds(64 + t * 16, 16)])
                orow_v[p, j, pl.ds(64, 16)] = exsp * ea_v[b, j, :]
                orow_v[p, j, pl.ds(80, 16)] = jnp.where(lane == 0, exsp, zero16)
            return carry
        lax.fori_loop(0, _K // 16, _group, 0)

    for b in range(_NB - 1):
        cb = b * _NTILE + sid
        pltpu.sync_copy(src_hbm.at[pl.ds(cb * _K, _K)], src_v.at[b])
        pltpu.sync_copy(dst_hbm.at[pl.ds(cb * _K, _K)], dst_v.at[b])
        _adjust_src(b)
        _issue(b, cb)
    _issue_idx(_NB - 1, (_NB - 1) * _NTILE + sid)

    def _outer(k2, carry):
        for b in range(_NB):
            k = k2 * _NB + b
            c = k * _NTILE + sid
            bl = (b - 1) % _NB
            cl = c + (_NB - 1) * _NTILE
            p = b & 1

            @pl.when(cl < _NCHUNK)
            def _():
                _wait_idx(bl)
                _adjust_src(bl)
                _issue(bl, cl)

            @pl.when(c < _NCHUNK)
            def _():
                _wait_gathers(b)

                if _EXP_SCATTER:
                    @pl.when(k >= 2)
                    def _():
                        _wait_scatter(p)
                    _copy_dst(b, p)
                if _EXP_COMPUTE:
                    _compute(b, p)
                if _EXP_SCATTER:
                    pltpu.async_copy(orow_v.at[p], acc_sh.at[dsc_v.at[p]],
                                     ssem, add=True)

                @pl.when(c + _NB * _NTILE < _NCHUNK)
                def _():
                    _issue_idx(b, c + _NB * _NTILE)
        return carry

    lax.fori_loop(0, -(-_ITERS // _NB), _outer, 0)
    if _EXP_SCATTER:
        _wait_scatter(0)
        _wait_scatter(1)
    plsc.subcore_barrier()

    for i in range(9):
        rows = pl.ds(tile0 + i * _K, _K)
        pltpu.sync_copy(acc_sh.at[rows], acc_hbm.at[cid, rows])

    @pl.when(sid < 15)
    def _():
        rows = pl.ds(tile0 + 9 * _K, 48)
        pltpu.sync_copy(acc_sh.at[rows], acc_hbm.at[cid, rows])

    @pl.when(sid == 15)
    def _():
        rows = pl.ds(tile0 + 9 * _K, _K)
        pltpu.sync_copy(acc_sh.at[rows], acc_hbm.at[cid, rows])


_edge_call = functools.partial(
    pl.kernel,
    mesh=_MESH,
    compiler_params=_SC_PARAMS,
    out_type=jax.ShapeDtypeStruct((2, _N, _DACC), jnp.float32),
    scratch_types=[
        pltpu.VMEM((_NB, _K), jnp.int32),
        pltpu.VMEM((_NB, _K), jnp.int32),
        pltpu.VMEM((_NB, _K, _HD), jnp.float32),
        pltpu.VMEM((_NB, _K, _HID + _ED), jnp.float32),
        pltpu.VMEM((_NB, _K, _ED), jnp.float32),
        pltpu.VMEM((2, _K, _DACC), jnp.float32),
        pltpu.VMEM((2, _K), jnp.int32),
        pltpu.VMEM((16, 17), jnp.float32),
        pltpu.VMEM_SHARED((_N, _DACC), jnp.float32),
        pltpu.SemaphoreType.DMA,
        pltpu.SemaphoreType.DMA,
        pltpu.SemaphoreType.DMA,
    ],
)(_edge_sc_body)


def _norm_h(acc, we_v, skip):
    out = []
    for h in range(_H):
        den = acc[h, :, 80:81] + 1e-16
        out.append((acc[h, :, :_HID] +
                    jnp.dot(acc[h, :, _HID:_HID + _ED],
                            we_v[h * _HID:(h + 1) * _HID, :].T,
                            preferred_element_type=jnp.float32)) / den
                   + skip[h])
    return jnp.maximum(jnp.concatenate(out, axis=1), 0.0)


def _post_body(acc_ref, skip_ref, we_ref, h_ref):
    h_ref[...] = _norm_h(acc_ref[...], we_ref[...], skip_ref[...])


def _post_call(acc, skip, we):
    blk = 2000
    return pl.pallas_call(
        _post_body,
        grid=(_N // blk,),
        in_specs=[pl.BlockSpec((2, blk, _DACC), lambda i: (0, i, 0)),
                  pl.BlockSpec((2, blk, _HID), lambda i: (0, i, 0)),
                  pl.BlockSpec(we.shape, lambda i: (0, 0))],
        out_specs=pl.BlockSpec((blk, _HD), lambda i: (i, 0)),
        out_shape=jax.ShapeDtypeStruct((_N, _HD), jnp.float32),
    )(acc, skip, we)


def _head_body(acc_ref, skip_ref, we_ref, batch_ref,
               wc1, bc1, wc2, bc2, out_ref):
    h = _norm_h(acc_ref[...], we_ref[...], skip_ref[...])

    b = batch_ref[...]
    gid = lax.broadcasted_iota(jnp.int32, (_G, _N), 0)
    m = jnp.where(b == gid, 1.0, 0.0)
    sums = jnp.dot(m, h, preferred_element_type=jnp.float32)
    cnt = jnp.sum(m, axis=1, keepdims=True)
    pooled = sums / jnp.maximum(cnt, 1.0)
    hid = jnp.maximum(jnp.dot(pooled, wc1[...].T,
                              preferred_element_type=jnp.float32) + bc1[...], 0.0)
    out_ref[...] = jnp.dot(hid, wc2[...].T,
                           preferred_element_type=jnp.float32) + bc2[...]


def _head_call(acc, skip, we, batch2d, wc1, bc1, wc2, bc2):
    return pl.pallas_call(
        _head_body,
        out_shape=jax.ShapeDtypeStruct((_G, _NCLS), jnp.float32),
    )(acc, skip, we, batch2d, wc1, bc1, wc2, bc2)


def _layer(h, src, dst, edge_attr, wq, bq, wk, bk, wv, bv, we, ws, bs):
    kv, qq, skip = _proj_call(h, wq, bq, wk, bk, wv, bv, we, ws, bs)
    kv2 = kv.reshape(_H * _N, _HD)
    qq2 = qq.reshape(_H * _N, _HID + _ED)
    acc = _edge_call(src, dst, edge_attr, kv2, qq2)
    return acc, skip


def kernel(x, edge_index, edge_attr, batch,
           Wq1, bq1, Wk1, bk1, Wv1, bv1, We1, Ws1, bs1,
           Wq2, bq2, Wk2, bk2, Wv2, bv2, We2, Ws2, bs2,
           Wc1, bc1, Wc2, bc2):
    src = edge_index[0].astype(jnp.int32)
    dst = edge_index[1].astype(jnp.int32)
    r2 = lambda b: b.reshape(1, -1)
    rb = lambda b: b.reshape(_H, 1, _HID)

    acc1, skip1 = _layer(x, src, dst, edge_attr,
                         Wq1, rb(bq1), Wk1, rb(bk1), Wv1, rb(bv1),
                         We1, Ws1, rb(bs1))
    h1 = _post_call(acc1, skip1, We1)
    acc2, skip2 = _layer(h1, src, dst, edge_attr,
                         Wq2, rb(bq2), Wk2, rb(bk2), Wv2, rb(bv2),
                         We2, Ws2, rb(bs2))
    return _head_call(acc2, skip2, We2,
                      batch.astype(jnp.int32).reshape(1, -1),
                      Wc1, r2(bc1), Wc2, r2(bc2))
